# baseline (speedup 1.0000x reference)
import functools

import jax
import jax.numpy as jnp
from jax.experimental import pallas as pl
from jax.experimental.pallas import tpu as pltpu

_ALPHA = 10.0
_K = 10
_BIG = 3.0e38


def _enc1_body(x_ref, w_ref, b_ref, g_ref, be_ref, o_ref):
    h = jnp.dot(w_ref[...], x_ref[...],
                preferred_element_type=jnp.float32)
    h = h + b_ref[...]
    m = jnp.mean(h, axis=1, keepdims=True)
    v = jnp.mean((h - m) ** 2, axis=1, keepdims=True)
    h = (h - m) / jnp.sqrt(v + 1e-5) * g_ref[...] + be_ref[...]
    o_ref[...] = jnp.maximum(h, 0.0)


def _enc2_body(x_ref, w2_ref, b2_ref, g2_ref, be2_ref, w3_ref, b3_ref, o_ref):
    h = jnp.dot(w2_ref[...], x_ref[...],
                preferred_element_type=jnp.float32)
    h = h + b2_ref[...]
    m = jnp.mean(h, axis=1, keepdims=True)
    v = jnp.mean((h - m) ** 2, axis=1, keepdims=True)
    h = (h - m) / jnp.sqrt(v + 1e-5) * g2_ref[...] + be2_ref[...]
    h = jnp.maximum(h, 0.0)
    lat = jnp.dot(w3_ref[...], h,
                  preferred_element_type=jnp.float32)
    lat = lat + b3_ref[...]
    norm = jnp.sqrt(jnp.sum(lat * lat, axis=0, keepdims=True))
    o_ref[...] = lat / jnp.maximum(norm, 1e-12)


def _knn_body(q_ref, k_ref, l_ref, o_ref):
    q = q_ref[...]
    keys = k_ref[...]
    lbl = l_ref[...]
    qn = jnp.sum(q * q, axis=0, keepdims=True)
    kn = jnp.sum(keys * keys, axis=1, keepdims=True)
    d = (qn - 2.0 * jnp.dot(keys, q,
                            preferred_element_type=jnp.float32)) + kn
    work = d
    for i in range(_K):
        t = jnp.min(work, axis=0, keepdims=True)
        if i < _K - 1:
            work = jnp.where(work <= t, _BIG, work)
    w = jnp.where(d <= t, jnp.exp(-_ALPHA * d), 0.0)
    num = jnp.sum(w * lbl, axis=0)
    den = jnp.sum(w, axis=0)
    o_ref[...] = num / (den + 1e-8)


def _im2col(x, ch):
    d, h, w = x.shape[1], x.shape[2], x.shape[3]
    xp = jnp.pad(x, ((0, 0), (1, 1), (1, 1), (1, 1)))
    cols = [xp[:, dz:dz + d, dy:dy + h, dx:dx + w]
            for dz in range(3) for dy in range(3) for dx in range(3)]
    return jnp.stack(cols).reshape(27 * ch, d * h * w)


def kernel(bg_prob, ed_prob, w1, b1, g1, be1, w2, b2, g2, be2, w3, b3,
           key_store, store_labels, context_mask, add_mode):
    B, _, D, H, W = bg_prob.shape
    N = B * D * H * W
    C = w3.shape[0]
    K = key_store.shape[0]

    x = jnp.concatenate([bg_prob, ed_prob], axis=1).reshape(2, D, H, W)
    x1 = _im2col(x, 2)
    w1m = jnp.transpose(w1, (2, 3, 4, 1, 0)).reshape(54, 16).T

    h1 = pl.pallas_call(
        _enc1_body,
        out_shape=jax.ShapeDtypeStruct((16, N), jnp.float32),
    )(x1, w1m, b1.reshape(16, 1), g1.reshape(16, 1), be1.reshape(16, 1))

    x2 = _im2col(h1.reshape(16, D, H, W), 16)
    w2m = jnp.transpose(w2, (2, 3, 4, 1, 0)).reshape(432, 32).T
    w3m = w3.reshape(C, 32)

    lat = pl.pallas_call(
        _enc2_body,
        out_shape=jax.ShapeDtypeStruct((C, N), jnp.float32),
    )(x2, w2m, b2.reshape(32, 1), g2.reshape(32, 1), be2.reshape(32, 1),
      w3m, b3.reshape(C, 1))

    R = 512
    prob = pl.pallas_call(
        _knn_body,
        grid=(N // R,),
        in_specs=[
            pl.BlockSpec((C, R), lambda i: (0, i)),
            pl.BlockSpec((K, C), lambda i: (0, 0)),
            pl.BlockSpec((K, 1), lambda i: (0, 0)),
        ],
        out_specs=pl.BlockSpec((R,), lambda i: (i,)),
        out_shape=jax.ShapeDtypeStruct((N,), jnp.float32),
        compiler_params=pltpu.CompilerParams(
            dimension_semantics=("parallel",)),
    )(lat, key_store, store_labels.reshape(K, 1))

    return prob.reshape(B, D, H, W)


# speedup vs baseline: 1.4044x; 1.4044x over previous
import jax
import jax.numpy as jnp
from jax.experimental import pallas as pl
from jax.experimental.pallas import tpu as pltpu

_ALPHA = 10.0
_K = 10
_BIG = 3.0e38


def _shift_cols(a, s, n):
    if s == 0:
        return a
    c = a.shape[0]
    if s > 0:
        return jnp.concatenate([a[:, s:], jnp.zeros((c, s), a.dtype)], axis=1)
    return jnp.concatenate([jnp.zeros((c, -s), a.dtype), a[:, :s]], axis=1)


def _enc_body(x1_ref, w1_ref, b1_ref, g1_ref, be1_ref,
              w29_ref, b2_ref, g2_ref, be2_ref, w3_ref, b3_ref, o_ref):
    n = x1_ref.shape[1]
    h = jnp.dot(w1_ref[...], x1_ref[...],
                preferred_element_type=jnp.float32)
    h = h + b1_ref[...]
    m = jnp.mean(h, axis=1, keepdims=True)
    v = jnp.mean((h - m) ** 2, axis=1, keepdims=True)
    h = (h - m) / jnp.sqrt(v + 1e-5) * g1_ref[...] + be1_ref[...]
    h = jnp.maximum(h, 0.0)

    col = jax.lax.broadcasted_iota(jnp.int32, (1, n), 1)
    xc = col % 32
    yc = (col // 32) % 32

    acc = jnp.zeros((32, n), jnp.float32)
    j = 0
    for ey in (-1, 0, 1):
        my = ((yc + ey) >= 0) & ((yc + ey) < 32)
        for ex in (-1, 0, 1):
            mask = (my & ((xc + ex) >= 0) & ((xc + ex) < 32)).astype(h.dtype)
            sxy = _shift_cols(h, 32 * ey + ex, n) * mask
            stk = jnp.concatenate(
                [_shift_cols(sxy, 1024 * ez, n) for ez in (-1, 0, 1)], axis=0)
            acc = acc + jnp.dot(w29_ref[32 * j:32 * (j + 1), :], stk,
                                preferred_element_type=jnp.float32)
            j += 1

    h2 = acc + b2_ref[...]
    m = jnp.mean(h2, axis=1, keepdims=True)
    v = jnp.mean((h2 - m) ** 2, axis=1, keepdims=True)
    h2 = (h2 - m) / jnp.sqrt(v + 1e-5) * g2_ref[...] + be2_ref[...]
    h2 = jnp.maximum(h2, 0.0)

    lat = jnp.dot(w3_ref[...], h2,
                  preferred_element_type=jnp.float32)
    lat = lat + b3_ref[...]
    norm = jnp.sqrt(jnp.sum(lat * lat, axis=0, keepdims=True))
    o_ref[...] = lat / jnp.maximum(norm, 1e-12)


def _knn_body(q_ref, k_ref, l_ref, o_ref):
    q = q_ref[...]
    keys = k_ref[...]
    lbl = l_ref[...]
    qn = jnp.sum(q * q, axis=0, keepdims=True)
    kn = jnp.sum(keys * keys, axis=1, keepdims=True)
    d = (qn - 2.0 * jnp.dot(keys, q,
                            preferred_element_type=jnp.float32)) + kn
    work = d
    for i in range(_K):
        t = jnp.min(work, axis=0, keepdims=True)
        if i < _K - 1:
            work = jnp.where(work <= t, _BIG, work)
    w = jnp.where(d <= t, jnp.exp(-_ALPHA * d), 0.0)
    num = jnp.sum(w * lbl, axis=0)
    den = jnp.sum(w, axis=0)
    o_ref[...] = num / (den + 1e-8)


def _im2col(x, ch):
    d, h, w = x.shape[1], x.shape[2], x.shape[3]
    xp = jnp.pad(x, ((0, 0), (1, 1), (1, 1), (1, 1)))
    cols = [xp[:, dz:dz + d, dy:dy + h, dx:dx + w]
            for dz in range(3) for dy in range(3) for dx in range(3)]
    return jnp.stack(cols).reshape(27 * ch, d * h * w)


def kernel(bg_prob, ed_prob, w1, b1, g1, be1, w2, b2, g2, be2, w3, b3,
           key_store, store_labels, context_mask, add_mode):
    B, _, D, H, W = bg_prob.shape
    N = B * D * H * W
    C = w3.shape[0]
    K = key_store.shape[0]

    x = jnp.concatenate([bg_prob, ed_prob], axis=1).reshape(2, D, H, W)
    x1 = _im2col(x, 2)
    x1 = jnp.pad(x1, ((0, 2), (0, 0)))
    w1m = jnp.transpose(w1, (2, 3, 4, 1, 0)).reshape(54, 16).T
    w1m = jnp.pad(w1m, ((0, 0), (0, 2)))

    w29 = jnp.transpose(w2, (3, 4, 0, 2, 1)).reshape(9 * 32, 48)
    w3m = w3.reshape(C, 32)

    lat = pl.pallas_call(
        _enc_body,
        out_shape=jax.ShapeDtypeStruct((C, N), jnp.float32),
    )(x1, w1m, b1.reshape(16, 1), g1.reshape(16, 1), be1.reshape(16, 1),
      w29, b2.reshape(32, 1), g2.reshape(32, 1), be2.reshape(32, 1),
      w3m, b3.reshape(C, 1))

    R = 512
    prob = pl.pallas_call(
        _knn_body,
        grid=(N // R,),
        in_specs=[
            pl.BlockSpec((C, R), lambda i: (0, i)),
            pl.BlockSpec((K, C), lambda i: (0, 0)),
            pl.BlockSpec((K, 1), lambda i: (0, 0)),
        ],
        out_specs=pl.BlockSpec((R,), lambda i: (i,)),
        out_shape=jax.ShapeDtypeStruct((N,), jnp.float32),
        compiler_params=pltpu.CompilerParams(
            dimension_semantics=("parallel",)),
    )(lat, key_store, store_labels.reshape(K, 1))

    return prob.reshape(B, D, H, W)
